# baseline (speedup 1.0000x reference)
import functools

import jax
import jax.numpy as jnp
from jax import lax
from jax.experimental import pallas as pl
from jax.experimental.pallas import tpu as pltpu
from jax.experimental.pallas import tpu_sc as plsc

B, N, L, D, V = 2, 128, 1024, 256, 21
VP = 32
LBLK = 256
NBLK = 8
LBLKT = 512
C = 128
NH = 32
NSLOT = 3


def _tc_body(seq_ref, qseq_ref, tab_ref, pos_ref, wt_ref, bias_ref, out_ref,
             base_ref):
    nb = pl.program_id(1)

    @pl.when(nb == 0)
    def _compute_base():
        qtok = qseq_ref[0, 0, :]
        oh_q = (qtok[None, :]
                == lax.broadcasted_iota(jnp.int32, (VP, LBLKT), 0)
                ).astype(jnp.float32)
        qe = lax.dot_general(oh_q, tab_ref[...], (((0,), (0,)), ((), ())),
                             precision=lax.Precision.HIGHEST)
        q = jnp.dot(qe, wt_ref[...], precision=lax.Precision.HIGHEST)
        base_ref[...] = pos_ref[...] + q + bias_ref[0, :]

    base = base_ref[...]
    for n in range(NBLK):
        tok = seq_ref[0, n, :]
        oh = (tok[None, :]
              == lax.broadcasted_iota(jnp.int32, (VP, LBLKT), 0)
              ).astype(jnp.float32)
        emb = lax.dot_general(oh, tab_ref[...], (((0,), (0,)), ((), ())),
                              precision=lax.Precision.HIGHEST)
        x = emb + base
        mu = jnp.mean(x, axis=-1, keepdims=True)
        xc = x - mu
        var = jnp.mean(xc * xc, axis=-1, keepdims=True)
        out_ref[0, n] = xc * lax.rsqrt(var + 1e-5)


def _tc_direct(msa_seq0, query_seq0, tab, pos_table, W, b):
    return pl.pallas_call(
        _tc_body,
        grid=(L // LBLKT, N // NBLK),
        in_specs=[
            pl.BlockSpec((1, NBLK, LBLKT), lambda lb, nb: (0, nb, lb)),
            pl.BlockSpec((1, 1, LBLKT), lambda lb, nb: (0, 0, lb)),
            pl.BlockSpec((VP, D), lambda lb, nb: (0, 0)),
            pl.BlockSpec((LBLKT, D), lambda lb, nb: (lb, 0)),
            pl.BlockSpec((D, D), lambda lb, nb: (0, 0)),
            pl.BlockSpec((1, D), lambda lb, nb: (0, 0)),
        ],
        out_specs=pl.BlockSpec((1, NBLK, LBLKT, D),
                               lambda lb, nb: (0, nb, lb, 0)),
        out_shape=jax.ShapeDtypeStruct((1, N, L, D), jnp.float32),
        scratch_shapes=[pltpu.VMEM((LBLKT, D), jnp.float32)],
        compiler_params=pltpu.CompilerParams(
            dimension_semantics=("arbitrary", "arbitrary"),
        ),
    )(msa_seq0, query_seq0.reshape(1, 1, L), tab, pos_table, W.T,
      b.reshape(1, D))


def _cand_body(qseq_ref, tab_ref, pos_ref, wt_ref, bias_ref, out_ref):
    qtok = qseq_ref[0, 0, :]
    oh = (qtok[None, :]
          == lax.broadcasted_iota(jnp.int32, (VP, LBLK), 0)).astype(jnp.float32)
    qe = lax.dot_general(oh, tab_ref[...], (((0,), (0,)), ((), ())),
                         precision=lax.Precision.HIGHEST)
    q = jnp.dot(qe, wt_ref[...], precision=lax.Precision.HIGHEST)
    base = pos_ref[...] + q + bias_ref[0, :]
    for v in range(V):
        x = base + tab_ref[v, :]
        mu = jnp.mean(x, axis=-1, keepdims=True)
        xc = x - mu
        var = jnp.mean(xc * xc, axis=-1, keepdims=True)
        out_ref[0, v] = xc * lax.rsqrt(var + 1e-5)


def _compute_cand(query_seq1, tab, pos_table, W, b):
    return pl.pallas_call(
        _cand_body,
        grid=(1, L // LBLK),
        in_specs=[
            pl.BlockSpec((1, 1, LBLK), lambda bi, lb: (bi, 0, lb)),
            pl.BlockSpec((VP, D), lambda bi, lb: (0, 0)),
            pl.BlockSpec((LBLK, D), lambda bi, lb: (lb, 0)),
            pl.BlockSpec((D, D), lambda bi, lb: (0, 0)),
            pl.BlockSpec((1, D), lambda bi, lb: (0, 0)),
        ],
        out_specs=pl.BlockSpec((1, V, LBLK, D), lambda bi, lb: (bi, 0, lb, 0)),
        out_shape=jax.ShapeDtypeStruct((1, V, L, D), jnp.float32),
    )(query_seq1.reshape(1, 1, L), tab, pos_table, W.T, b.reshape(1, D))


def _sc_body(seq_hbm, cand_hbm, out_hbm,
             seq_v, idx0, idx1, idx2, gb0, gb1, gb2,
             gsem0, gsem1, gsem2, osem0, osem1, osem2):
    nc = 2
    wid = lax.axis_index("s") * nc + lax.axis_index("c")
    l0 = (wid // 4) * C
    n0 = (wid % 4) * NH

    pltpu.sync_copy(seq_hbm.at[pl.ds(n0, NH), pl.ds(l0, C)], seq_v)

    lane = lax.iota(jnp.int32, 16)
    lbase = [l0 + g * 16 + lane for g in range(C // 16)]

    idxs = (idx0, idx1, idx2)
    gbufs = (gb0, gb1, gb2)
    gsems = (gsem0, gsem1, gsem2)
    osems = (osem0, osem1, osem2)

    def start_gather(k, nl, first):
        @pl.when(jnp.logical_not(first))
        def _scatter_done():
            pltpu.make_async_copy(
                gbufs[k], out_hbm.at[pl.ds(0, C)], osems[k]).wait()

        for g in range(C // 16):
            tok = seq_v[nl, pl.ds(g * 16, 16)]
            idxs[k][pl.ds(g * 16, 16)] = lbase[g] + tok * L
        return pltpu.async_copy(cand_hbm.at[idxs[k]], gbufs[k], gsems[k])

    def start_scatter(k, nl, gather):
        gather.wait()
        row0 = (n0 + nl) * L + l0
        pltpu.async_copy(gbufs[k], out_hbm.at[pl.ds(row0, C)], osems[k])

    def n_iter(n3, _):
        gathers = [start_gather(k, n3 * NSLOT + k, n3 == 0)
                   for k in range(NSLOT)]
        for k in range(NSLOT):
            start_scatter(k, n3 * NSLOT + k, gathers[k])
        return 0

    nfull = NH // NSLOT
    lax.fori_loop(0, nfull, n_iter, 0)
    for k in range(NH - nfull * NSLOT):
        nl = nfull * NSLOT + k
        start_scatter(k, nl, start_gather(k, nl, False))
    for k in range(NSLOT):
        pltpu.make_async_copy(
            gbufs[k], out_hbm.at[pl.ds(0, C)], osems[k]).wait()


@jax.jit
def kernel(msa_seq, mask, query_seq, msa_table, pos_table, W, b, gamma, beta):
    tab = jnp.zeros((VP, D), jnp.float32).at[:V].set(msa_table)

    cand = _compute_cand(query_seq[1:2], tab, pos_table, W, b)
    sc = functools.partial(
        pl.kernel,
        mesh=plsc.VectorSubcoreMesh(core_axis_name="c", subcore_axis_name="s"),
        out_type=jax.ShapeDtypeStruct((N * L, D), jnp.float32),
        scratch_types=(
            [pltpu.VMEM((NH, C), jnp.int32)]
            + [pltpu.VMEM((C,), jnp.int32) for _ in range(NSLOT)]
            + [pltpu.VMEM((C, D), jnp.float32) for _ in range(NSLOT)]
            + [pltpu.SemaphoreType.DMA for _ in range(2 * NSLOT)]
        ),
    )(_sc_body)
    out1 = sc(msa_seq[1], cand.reshape(V * L, D)).reshape(1, N, L, D)

    out0 = _tc_direct(msa_seq[0:1], query_seq[0:1], tab, pos_table, W, b)
    return jnp.concatenate([out0, out1], axis=0)
